# baseline (speedup 1.0000x reference)
import functools

import jax
import jax.numpy as jnp
from jax import lax
from jax.experimental import pallas as pl
from jax.experimental.pallas import tpu as pltpu
from jax.experimental.pallas import tpu_sc as plsc

_K = 3
_NITERS = 16
_D = 144
_NCH = 16
_L16 = 16

_NW = 32
_RPW_SC = 32
_NCHUNK = _RPW_SC // _L16
_ROWS_SC = _NW * _RPW_SC
_TC_BLOCKS = 41


def _sc_wos_call(u_pad, mask, weight, bias_row):
    mesh = plsc.VectorSubcoreMesh(core_axis_name="c", subcore_axis_name="s")

    @functools.partial(
        pl.kernel,
        mesh=mesh,
        out_type=jax.ShapeDtypeStruct((_NW, _NCH, _RPW_SC), jnp.float32),
        compiler_params=pltpu.CompilerParams(needs_layout_passes=False,
                                             use_tc_tiling_on_sc=False),
        scratch_types=[
            pltpu.VMEM((_RPW_SC, _D), jnp.float32),
            pltpu.VMEM((_NCH, 2 * _D), jnp.float32),
            pltpu.VMEM((_NCH, 2 * _D), jnp.float32),
            pltpu.VMEM((1, _L16), jnp.float32),
            pltpu.VMEM((_D, _L16), jnp.float32),
            pltpu.VMEM((_D, _L16), jnp.float32),
            pltpu.VMEM((_D, _L16), jnp.float32),
            pltpu.VMEM((_D, _L16), jnp.float32),
            pltpu.VMEM((_D, _L16), jnp.float32),
            pltpu.VMEM((_D, _L16), jnp.float32),
            pltpu.VMEM((_NCH, _RPW_SC), jnp.float32),
        ],
    )
    def k(u_hbm, mask_hbm, w_hbm, bias_hbm, out_hbm,
          u_v, mask_v, w_v, bias_v, mp_v, mm_v, wp_v, wm_v, mxp_v, mxm_v, out_v):
        wid = lax.axis_index("s") * 2 + lax.axis_index("c")
        pltpu.sync_copy(u_hbm.at[wid], u_v)
        pltpu.sync_copy(mask_hbm, mask_v)
        pltpu.sync_copy(w_hbm, w_v)
        pltpu.sync_copy(bias_hbm, bias_v)

        lane = lax.iota(jnp.int32, _L16)
        zf = jnp.zeros((_L16,), jnp.float32)
        zi = jnp.zeros((_L16,), jnp.int32)
        inf = jnp.full((_L16,), jnp.inf, jnp.float32)

        def chan_body(c, _):
            c_idx = zi + c
            b = plsc.load_gather(bias_v, [zi, c_idx])

            def splat_body(d, didx):
                mp_v[d] = plsc.load_gather(mask_v, [c_idx, didx])
                mm_v[d] = plsc.load_gather(mask_v, [c_idx, didx + _D])
                wp_v[d] = plsc.load_gather(w_v, [c_idx, didx])
                wm_v[d] = plsc.load_gather(w_v, [c_idx, didx + _D])
                return didx + 1
            lax.fori_loop(0, _D, splat_body, zi, unroll=2)

            def chunk_body(ch, _):
                rows = ch * _L16 + lane

                def pre_body(d, carry):
                    didx, mn, mx = carry
                    ug = plsc.load_gather(u_v, [rows, didx])
                    a = ug + mp_v[d]
                    bb = mm_v[d] - ug
                    mxp_v[d] = a
                    mxm_v[d] = bb
                    mn = jnp.minimum(mn, jnp.minimum(a, bb))
                    mx = jnp.maximum(mx, jnp.maximum(a, bb))
                    return didx + 1, mn, mx
                _, lo0, hi0 = lax.fori_loop(0, _D, pre_body, (zi, inf, -inf),
                                            unroll=2)

                def bis_body(_, lohi):
                    lo, hi = lohi
                    t = 0.5 * (lo + hi)

                    def acc_body(d4, acc):
                        d = d4 * 4
                        for q in range(4):
                            acc = acc + jnp.where(mxp_v[d + q] >= t, wp_v[d + q], zf)
                            acc = acc + jnp.where(mxm_v[d + q] >= t, wm_v[d + q], zf)
                        return acc
                    g = lax.fori_loop(0, _D // 4, acc_body, zf)
                    le = g <= b
                    return jnp.where(le, lo, t), jnp.where(le, t, hi)
                _, hi = lax.fori_loop(0, _NITERS, bis_body, (lo0, hi0))

                def ext_body(d4, ans):
                    d = d4 * 4
                    for q in range(4):
                        a = mxp_v[d + q]
                        bb = mxm_v[d + q]
                        ans = jnp.minimum(ans, jnp.where(a >= hi, a, inf))
                        ans = jnp.minimum(ans, jnp.where(bb >= hi, bb, inf))
                    return ans
                ans = lax.fori_loop(0, _D // 4, ext_body, inf)
                out_v[c, pl.ds(ch * _L16, _L16)] = ans
                return 0
            lax.fori_loop(0, _NCHUNK, chunk_body, 0)
            return 0

        lax.fori_loop(0, _NCH, chan_body, 0)
        pltpu.sync_copy(out_v, out_hbm.at[wid])

    return k(u_pad, mask, weight, bias_row)


def _tc_body(u_ref, mp_ref, mm_ref, wp_ref, wm_ref, bias_ref, out_ref):
    c = pl.program_id(1)
    u = u_ref[...]
    mp = mp_ref[0]
    mm = mm_ref[0]
    wp = wp_ref[0]
    wm = wm_ref[0]
    b = bias_ref[c, 0]

    mxp = u + mp
    mxm = mm - u

    hi0 = jnp.maximum(jnp.max(mxp, axis=0, keepdims=True),
                      jnp.max(mxm, axis=0, keepdims=True))
    lo0 = jnp.minimum(jnp.min(mxp, axis=0, keepdims=True),
                      jnp.min(mxm, axis=0, keepdims=True))

    h = mxp.shape[0] // 2

    def body(_, carry):
        lo, hi = carry
        t = 0.5 * (lo + hi)
        g = ((jnp.sum(jnp.where(mxp[:h] >= t, wp[:h], 0.0), axis=0, keepdims=True)
              + jnp.sum(jnp.where(mxp[h:] >= t, wp[h:], 0.0), axis=0, keepdims=True))
             + (jnp.sum(jnp.where(mxm[:h] >= t, wm[:h], 0.0), axis=0, keepdims=True)
                + jnp.sum(jnp.where(mxm[h:] >= t, wm[h:], 0.0), axis=0, keepdims=True)))
        le = g <= b
        return jnp.where(le, lo, t), jnp.where(le, t, hi)

    lo, hi = jax.lax.fori_loop(0, _NITERS, body, (lo0, hi0))

    inf = jnp.float32(jnp.inf)
    ap = jnp.min(jnp.where(mxp >= hi, mxp, inf), axis=0, keepdims=True)
    am = jnp.min(jnp.where(mxm >= hi, mxm, inf), axis=0, keepdims=True)
    ans = jnp.minimum(ap, am)
    ans = jnp.where(jnp.isfinite(ans), ans, hi0)
    out_ref[...] = ans[None]


def _tc_wos_call(uT, maskp, maskm, wp, wm, bias, n_rows, n_chan, d, n_blocks):
    rb = n_rows // n_blocks
    return pl.pallas_call(
        _tc_body,
        grid=(n_blocks, n_chan),
        in_specs=[
            pl.BlockSpec((d, rb), lambda r, c: (0, r)),
            pl.BlockSpec((1, d, 1), lambda r, c: (c, 0, 0)),
            pl.BlockSpec((1, d, 1), lambda r, c: (c, 0, 0)),
            pl.BlockSpec((1, d, 1), lambda r, c: (c, 0, 0)),
            pl.BlockSpec((1, d, 1), lambda r, c: (c, 0, 0)),
            pl.BlockSpec(memory_space=pltpu.SMEM),
        ],
        out_specs=pl.BlockSpec((1, 1, rb), lambda r, c: (c, 0, r)),
        out_shape=jax.ShapeDtypeStruct((n_chan, 1, n_rows), jnp.float32),
    )(uT, maskp, maskm, wp, wm, bias)


@jax.jit
def kernel(x, mask, weight, bias):
    b_, c_, h_, w_ = x.shape
    d = c_ * _K * _K
    nc = mask.shape[0]
    l = h_ * w_
    n = b_ * l

    xp = jnp.pad(x, ((0, 0), (0, 0), (1, 1), (1, 1)))
    patches = [xp[:, :, i:i + h_, j:j + w_] for i in range(_K) for j in range(_K)]
    u = jnp.stack(patches, axis=2).reshape(b_, d, l)
    uT = jnp.transpose(u, (1, 0, 2)).reshape(d, n)

    u_sc = jnp.transpose(uT[:, :_ROWS_SC]).reshape(_NW, _RPW_SC, d)
    y_sc = _sc_wos_call(u_sc, mask, weight, bias.reshape(1, nc))
    y_sc = jnp.transpose(y_sc, (1, 0, 2)).reshape(nc, _ROWS_SC)

    maskp = mask[:, :d, None]
    maskm = mask[:, d:, None]
    wp = weight[:, :d, None]
    wm = weight[:, d:, None]
    y_tc = _tc_wos_call(uT[:, _ROWS_SC:], maskp, maskm, wp, wm, bias,
                        n - _ROWS_SC, nc, d, _TC_BLOCKS)

    y = jnp.concatenate([y_sc, y_tc[:, 0, :]], axis=1)
    out = y.reshape(nc, b_, l).transpose(1, 0, 2).reshape(b_, nc, h_, w_)
    return out


# speedup vs baseline: 3.9038x; 3.9038x over previous
import functools

import jax
import jax.numpy as jnp
import numpy as np
from jax.experimental import pallas as pl
from jax.experimental.pallas import tpu as pltpu

_K = 3
_NITERS = 16


def _wos_body(u_ref, mp_ref, mm_ref, wp_ref, wm_ref, bias_ref, out_ref):
    c = pl.program_id(1)
    u = u_ref[...]
    mp = mp_ref[0]
    mm = mm_ref[0]
    wp = wp_ref[0]
    wm = wm_ref[0]
    b = bias_ref[c, 0]

    mxp = u + mp
    mxm = mm - u

    hi0 = jnp.maximum(jnp.max(mxp, axis=0, keepdims=True),
                      jnp.max(mxm, axis=0, keepdims=True))
    lo0 = jnp.minimum(jnp.min(mxp, axis=0, keepdims=True),
                      jnp.min(mxm, axis=0, keepdims=True))

    def body(_, carry):
        lo, hi = carry
        t = 0.5 * (lo + hi)
        gp = jnp.sum(jnp.where(mxp >= t, wp, 0.0), axis=0, keepdims=True)
        gm = jnp.sum(jnp.where(mxm >= t, wm, 0.0), axis=0, keepdims=True)
        le = (gp + gm) <= b
        return jnp.where(le, lo, t), jnp.where(le, t, hi)

    lo, hi = jax.lax.fori_loop(0, _NITERS, body, (lo0, hi0))

    inf = jnp.float32(jnp.inf)
    ap = jnp.min(jnp.where(mxp >= hi, mxp, inf), axis=0, keepdims=True)
    am = jnp.min(jnp.where(mxm >= hi, mxm, inf), axis=0, keepdims=True)
    ans = jnp.minimum(ap, am)
    ans = jnp.where(jnp.isfinite(ans), ans, hi0)
    out_ref[...] = ans[None]


def _wos_select(uT, maskp, maskm, wp, wm, bias, n_rows, n_chan, d):
    n_blocks = 1
    rb = n_rows // n_blocks

    return pl.pallas_call(
        _wos_body,
        grid=(n_blocks, n_chan),
        in_specs=[
            pl.BlockSpec((d, rb), lambda r, c: (0, r)),
            pl.BlockSpec((1, d, 1), lambda r, c: (c, 0, 0)),
            pl.BlockSpec((1, d, 1), lambda r, c: (c, 0, 0)),
            pl.BlockSpec((1, d, 1), lambda r, c: (c, 0, 0)),
            pl.BlockSpec((1, d, 1), lambda r, c: (c, 0, 0)),
            pl.BlockSpec(memory_space=pltpu.SMEM),
        ],
        out_specs=pl.BlockSpec((1, 1, rb), lambda r, c: (c, 0, r)),
        out_shape=jax.ShapeDtypeStruct((n_chan, 1, n_rows), jnp.float32),
    )(uT, maskp, maskm, wp, wm, bias)


@jax.jit
def kernel(x, mask, weight, bias):
    b_, c_, h_, w_ = x.shape
    d = c_ * _K * _K
    nc = mask.shape[0]
    l = h_ * w_
    n = b_ * l

    xp = jnp.pad(x, ((0, 0), (0, 0), (1, 1), (1, 1)))
    patches = [xp[:, :, i:i + h_, j:j + w_] for i in range(_K) for j in range(_K)]
    u = jnp.stack(patches, axis=2).reshape(b_, d, l)
    uT = jnp.transpose(u, (1, 0, 2)).reshape(d, n)

    maskp = mask[:, :d, None]
    maskm = mask[:, d:, None]
    wp = weight[:, :d, None]
    wm = weight[:, d:, None]

    y = _wos_select(uT, maskp, maskm, wp, wm, bias, n, nc, d)
    out = y.reshape(nc, b_, l).transpose(1, 0, 2).reshape(b_, nc, h_, w_)
    return out


# speedup vs baseline: 3.9204x; 1.0043x over previous
import functools

import jax
import jax.numpy as jnp
import numpy as np
from jax.experimental import pallas as pl
from jax.experimental.pallas import tpu as pltpu

_K = 3
_NITERS = 14


def _wos_body(u_ref, mp_ref, mm_ref, wp_ref, wm_ref, bias_ref, out_ref):
    c = pl.program_id(1)
    u = u_ref[...]
    mp = mp_ref[0]
    mm = mm_ref[0]
    wp = wp_ref[0]
    wm = wm_ref[0]
    b = bias_ref[c, 0]

    mxp = u + mp
    mxm = mm - u

    hi0 = jnp.maximum(jnp.max(mxp, axis=0, keepdims=True),
                      jnp.max(mxm, axis=0, keepdims=True))
    lo0 = jnp.minimum(jnp.min(mxp, axis=0, keepdims=True),
                      jnp.min(mxm, axis=0, keepdims=True))

    def body(_, carry):
        lo, hi = carry
        t = 0.5 * (lo + hi)
        g = jnp.sum(jnp.where(mxp >= t, wp, 0.0) + jnp.where(mxm >= t, wm, 0.0),
                    axis=0, keepdims=True)
        le = g <= b
        return jnp.where(le, lo, t), jnp.where(le, t, hi)

    lo, hi = jax.lax.fori_loop(0, _NITERS, body, (lo0, hi0))

    inf = jnp.float32(jnp.inf)
    ap = jnp.min(jnp.where(mxp >= hi, mxp, inf), axis=0, keepdims=True)
    am = jnp.min(jnp.where(mxm >= hi, mxm, inf), axis=0, keepdims=True)
    ans = jnp.minimum(ap, am)
    ans = jnp.where(jnp.isfinite(ans), ans, hi0)
    out_ref[...] = ans[None]


def _wos_select(uT, maskp, maskm, wp, wm, bias, n_rows, n_chan, d):
    n_blocks = 1
    rb = n_rows // n_blocks

    return pl.pallas_call(
        _wos_body,
        grid=(n_blocks, n_chan),
        in_specs=[
            pl.BlockSpec((d, rb), lambda r, c: (0, r)),
            pl.BlockSpec((1, d, 1), lambda r, c: (c, 0, 0)),
            pl.BlockSpec((1, d, 1), lambda r, c: (c, 0, 0)),
            pl.BlockSpec((1, d, 1), lambda r, c: (c, 0, 0)),
            pl.BlockSpec((1, d, 1), lambda r, c: (c, 0, 0)),
            pl.BlockSpec(memory_space=pltpu.SMEM),
        ],
        out_specs=pl.BlockSpec((1, 1, rb), lambda r, c: (c, 0, r)),
        out_shape=jax.ShapeDtypeStruct((n_chan, 1, n_rows), jnp.float32),
    )(uT, maskp, maskm, wp, wm, bias)


@jax.jit
def kernel(x, mask, weight, bias):
    b_, c_, h_, w_ = x.shape
    d = c_ * _K * _K
    nc = mask.shape[0]
    l = h_ * w_
    n = b_ * l

    xp = jnp.pad(x, ((0, 0), (0, 0), (1, 1), (1, 1)))
    patches = [xp[:, :, i:i + h_, j:j + w_] for i in range(_K) for j in range(_K)]
    u = jnp.stack(patches, axis=2).reshape(b_, d, l)
    uT = jnp.transpose(u, (1, 0, 2)).reshape(d, n)

    maskp = mask[:, :d, None]
    maskm = mask[:, d:, None]
    wp = weight[:, :d, None]
    wm = weight[:, d:, None]

    y = _wos_select(uT, maskp, maskm, wp, wm, bias, n, nc, d)
    out = y.reshape(nc, b_, l).transpose(1, 0, 2).reshape(b_, nc, h_, w_)
    return out
